# baseline (speedup 1.0000x reference)
import jax
import jax.numpy as jnp
from jax import lax
from jax.experimental import pallas as pl
from jax.experimental.pallas import tpu as pltpu
from jax.experimental.pallas import tpu_sc as plsc

_K_BY_HEAD = (10, 20, 40, 500)
_NW = 32
_G = 16


def _sc_body(att_hbm, out_hbm, in_buf, out_buf, hist,
             sem_in0, sem_in1, sem_out):
    total_rows, n = att_hbm.shape
    rows_per_w = total_rows // _NW
    ngroups = rows_per_w // _G
    wid = lax.axis_index("c") * 16 + lax.axis_index("s")
    row0 = wid * rows_per_w
    head = lax.rem(row0 // n, 4)
    k0 = jnp.where(
        head == 0, _K_BY_HEAD[0],
        jnp.where(head == 1, _K_BY_HEAD[1],
                  jnp.where(head == 2, _K_BY_HEAD[2], _K_BY_HEAD[3])))
    k0 = jnp.minimum(k0, n).astype(jnp.int32)
    sems_in = (sem_in0, sem_in1)

    iota = lax.iota(jnp.int32, 16)
    ones16 = jnp.ones((16,), jnp.int32)
    zeros16 = jnp.zeros((16,), jnp.int32)
    k16 = jnp.broadcast_to(k0, (16,))

    def in_dma(g, sl):
        return pltpu.make_async_copy(
            att_hbm.at[pl.ds(row0 + g * _G, _G)], in_buf.at[sl], sems_in[sl])

    def out_dma(g):
        return pltpu.make_async_copy(
            out_buf, out_hbm.at[pl.ds(row0 + g * _G, _G)], sem_out)

    def scan_hist(rem_k16):
        def sweep(j, carry):
            suffix, cnt, gt = carry
            b = 255 - j
            h = hist[pl.ds(b * 16, 16)]
            hist[pl.ds(b * 16, 16)] = zeros16
            suffix = suffix + h
            mask = suffix >= rem_k16
            cnt = cnt + jnp.where(mask, 1, 0)
            gt = gt + jnp.where(mask, 0, h)
            return suffix, cnt, gt

        _, cnt, gt = lax.fori_loop(0, 256, sweep, (zeros16, zeros16, zeros16),
                                   unroll=8)
        return cnt - 1, gt

    def per_group(g, sl):
        slv = jnp.full((16,), sl, jnp.int32)
        in_dma(g, sl).wait()

        def pass0(c, m16):
            col = jnp.broadcast_to(c, (16,))
            x = plsc.load_gather(in_buf, [slv, iota, col])
            b = lax.bitcast_convert_type(x, jnp.int32)
            key = jnp.where(b >= 0, b, b ^ jnp.int32(0x7FFFFFFF))
            plsc.store_scatter(in_buf, [slv, iota, col],
                               lax.bitcast_convert_type(key, jnp.float32))
            bkt = (key >> 24) + 128
            plsc.addupdate_scatter(hist, [bkt * 16 + iota], ones16)
            return jnp.maximum(m16, x)

        m16 = lax.fori_loop(0, n, pass0, jnp.full((16,), -3.4e38, jnp.float32),
                            unroll=8)
        bstar, gt = scan_hist(k16)
        prefix = bstar - 128
        rem_k = k16 - gt

        def radix_pass(shift, prefix, rem_k):
            def body(c, _):
                col = jnp.broadcast_to(c, (16,))
                key = plsc.load_gather(in_buf, [slv, iota, col])
                key = lax.bitcast_convert_type(key, jnp.int32)
                match = (key >> (shift + 8)) == prefix
                bkt = (key >> shift) & 0xFF
                plsc.addupdate_scatter(hist, [bkt * 16 + iota], ones16,
                                       mask=match)
                return 0

            lax.fori_loop(0, n, body, 0, unroll=8)
            b, gt = scan_hist(rem_k)
            return (prefix << 8) | b, rem_k - gt

        prefix, rem_k = radix_pass(16, prefix, rem_k)
        prefix, rem_k = radix_pass(8, prefix, rem_k)
        t16, _ = radix_pass(0, prefix, rem_k)

        @pl.when(g >= 1)
        def _():
            out_dma(g - 1).wait()

        def passA(c, z16):
            col = jnp.broadcast_to(c, (16,))
            key = plsc.load_gather(in_buf, [slv, iota, col])
            key = lax.bitcast_convert_type(key, jnp.int32)
            bb = jnp.where(key >= 0, key, key ^ jnp.int32(0x7FFFFFFF))
            x = lax.bitcast_convert_type(bb, jnp.float32)
            e = jnp.exp(x - m16)
            em = jnp.where(key >= t16, e, 0.0)
            plsc.store_scatter(out_buf, [iota, col], em)
            return z16 + em

        z16 = lax.fori_loop(0, n, passA, jnp.zeros((16,), jnp.float32),
                            unroll=8)
        invz = jnp.ones((16,), jnp.float32) / z16

        def passB(c, _):
            col = jnp.broadcast_to(c, (16,))
            em = plsc.load_gather(out_buf, [iota, col])
            plsc.store_scatter(out_buf, [iota, col], em * invz)
            return 0

        lax.fori_loop(0, n, passB, 0, unroll=8)
        out_dma(g).start()

    def zhist(v, _):
        hist[pl.ds(v * 16, 16)] = zeros16
        return 0
    lax.fori_loop(0, 256, zhist, 0, unroll=8)

    in_dma(0, 0).start()
    in_dma(1, 1).start()

    def per_pair(p, _):
        for sl in (0, 1):
            g = 2 * p + sl
            per_group(g, sl)

            @pl.when(g + 2 < ngroups)
            def _():
                in_dma(g + 2, sl).start()
        return 0

    lax.fori_loop(0, ngroups // 2, per_pair, 0)
    out_dma(ngroups - 1).wait()


def kernel(attention):
    bsz, heads, n, _ = attention.shape
    att2 = attention.reshape(bsz * heads * n, n)
    mesh = plsc.VectorSubcoreMesh(core_axis_name="c", subcore_axis_name="s")
    out2 = pl.kernel(
        _sc_body,
        out_type=jax.ShapeDtypeStruct(att2.shape, att2.dtype),
        mesh=mesh,
        compiler_params=pltpu.CompilerParams(needs_layout_passes=False),
        scratch_types=[
            pltpu.VMEM((2, _G, n), jnp.float32),
            pltpu.VMEM((_G, n), jnp.float32),
            pltpu.VMEM((256 * 16,), jnp.int32),
            pltpu.SemaphoreType.DMA,
            pltpu.SemaphoreType.DMA,
            pltpu.SemaphoreType.DMA,
        ],
    )(att2)
    return out2.reshape(attention.shape)


# speedup vs baseline: 2.2680x; 2.2680x over previous
import jax
import jax.numpy as jnp
from jax import lax
from jax.experimental import pallas as pl
from jax.experimental.pallas import tpu as pltpu
from jax.experimental.pallas import tpu_sc as plsc

_K_BY_HEAD = (10, 20, 40, 500)
_NW = 32
_G = 16


def _sc_body(att_hbm, out_hbm, in_buf, out_buf, hist,
             sem_in0, sem_in1, sem_out):
    total_rows, n = att_hbm.shape
    rows_per_w = total_rows // _NW
    ngroups = rows_per_w // _G
    wid = lax.axis_index("c") * 16 + lax.axis_index("s")
    row0 = wid * rows_per_w
    head = lax.rem(row0 // n, 4)
    k0 = jnp.where(
        head == 0, _K_BY_HEAD[0],
        jnp.where(head == 1, _K_BY_HEAD[1],
                  jnp.where(head == 2, _K_BY_HEAD[2], _K_BY_HEAD[3])))
    k0 = jnp.minimum(k0, n).astype(jnp.int32)
    sems_in = (sem_in0, sem_in1)

    iota = lax.iota(jnp.int32, 16)
    ones16 = jnp.ones((16,), jnp.int32)
    zeros16 = jnp.zeros((16,), jnp.int32)
    k16 = jnp.broadcast_to(k0, (16,))

    def in_dma(g, sl):
        return pltpu.make_async_copy(
            att_hbm.at[pl.ds(row0 + g * _G, _G)], in_buf.at[sl], sems_in[sl])

    def out_dma(g):
        return pltpu.make_async_copy(
            out_buf, out_hbm.at[pl.ds(row0 + g * _G, _G)], sem_out)

    def scan_hist(rem_k16):
        def sweep(j, carry):
            suffix, cnt, gt = carry
            b = 255 - j
            h = hist[pl.ds(b * 16, 16)]
            hist[pl.ds(b * 16, 16)] = zeros16
            suffix = suffix + h
            mask = suffix >= rem_k16
            cnt = cnt + jnp.where(mask, 1, 0)
            gt = gt + jnp.where(mask, 0, h)
            return suffix, cnt, gt

        _, cnt, gt = lax.fori_loop(0, 256, sweep, (zeros16, zeros16, zeros16),
                                   unroll=8)
        return cnt - 1, gt

    def per_group(g, sl):
        slv = jnp.full((16,), sl, jnp.int32)
        in_dma(g, sl).wait()

        def pass0(c, m16):
            col = (iota + c) & (n - 1)
            x = plsc.load_gather(in_buf, [slv, iota, col])
            b = lax.bitcast_convert_type(x, jnp.int32)
            key = jnp.where(b >= 0, b, b ^ jnp.int32(0x7FFFFFFF))
            plsc.store_scatter(in_buf, [slv, iota, col],
                               lax.bitcast_convert_type(key, jnp.float32))
            bkt = (key >> 24) + 128
            plsc.addupdate_scatter(hist, [bkt * 16 + iota], ones16)
            return jnp.maximum(m16, x)

        m16 = lax.fori_loop(0, n, pass0, jnp.full((16,), -3.4e38, jnp.float32),
                            unroll=8)
        bstar, gt = scan_hist(k16)
        prefix = bstar - 128
        rem_k = k16 - gt

        def radix_pass(shift, prefix, rem_k):
            def body(c, _):
                col = (iota + c) & (n - 1)
                key = plsc.load_gather(in_buf, [slv, iota, col])
                key = lax.bitcast_convert_type(key, jnp.int32)
                match = (key >> (shift + 8)) == prefix
                bkt = (key >> shift) & 0xFF
                plsc.addupdate_scatter(hist, [bkt * 16 + iota], ones16,
                                       mask=match)
                return 0

            lax.fori_loop(0, n, body, 0, unroll=8)
            b, gt = scan_hist(rem_k)
            return (prefix << 8) | b, rem_k - gt

        prefix, rem_k = radix_pass(16, prefix, rem_k)
        prefix, rem_k = radix_pass(8, prefix, rem_k)
        t16, _ = radix_pass(0, prefix, rem_k)

        @pl.when(g >= 1)
        def _():
            out_dma(g - 1).wait()

        def passA(c, z16):
            col = (iota + c) & (n - 1)
            key = plsc.load_gather(in_buf, [slv, iota, col])
            key = lax.bitcast_convert_type(key, jnp.int32)
            bb = jnp.where(key >= 0, key, key ^ jnp.int32(0x7FFFFFFF))
            x = lax.bitcast_convert_type(bb, jnp.float32)
            e = jnp.exp(x - m16)
            em = jnp.where(key >= t16, e, 0.0)
            plsc.store_scatter(out_buf, [iota, col], em)
            return z16 + em

        z16 = lax.fori_loop(0, n, passA, jnp.zeros((16,), jnp.float32),
                            unroll=8)
        invz = jnp.ones((16,), jnp.float32) / z16

        def passB(c, _):
            col = (iota + c) & (n - 1)
            em = plsc.load_gather(out_buf, [iota, col])
            plsc.store_scatter(out_buf, [iota, col], em * invz)
            return 0

        lax.fori_loop(0, n, passB, 0, unroll=8)
        out_dma(g).start()

    def zhist(v, _):
        hist[pl.ds(v * 16, 16)] = zeros16
        return 0
    lax.fori_loop(0, 256, zhist, 0, unroll=8)

    in_dma(0, 0).start()
    in_dma(1, 1).start()

    def per_pair(p, _):
        for sl in (0, 1):
            g = 2 * p + sl
            per_group(g, sl)

            @pl.when(g + 2 < ngroups)
            def _():
                in_dma(g + 2, sl).start()
        return 0

    lax.fori_loop(0, ngroups // 2, per_pair, 0)
    out_dma(ngroups - 1).wait()


def kernel(attention):
    bsz, heads, n, _ = attention.shape
    att2 = attention.reshape(bsz * heads * n, n)
    mesh = plsc.VectorSubcoreMesh(core_axis_name="c", subcore_axis_name="s")
    out2 = pl.kernel(
        _sc_body,
        out_type=jax.ShapeDtypeStruct(att2.shape, att2.dtype),
        mesh=mesh,
        compiler_params=pltpu.CompilerParams(needs_layout_passes=False),
        scratch_types=[
            pltpu.VMEM((2, _G, n), jnp.float32),
            pltpu.VMEM((_G, n), jnp.float32),
            pltpu.VMEM((256 * 16,), jnp.int32),
            pltpu.SemaphoreType.DMA,
            pltpu.SemaphoreType.DMA,
            pltpu.SemaphoreType.DMA,
        ],
    )(att2)
    return out2.reshape(attention.shape)


# speedup vs baseline: 8.0150x; 3.5340x over previous
import jax
import jax.numpy as jnp
from jax import lax
from jax.experimental import pallas as pl
from jax.experimental.pallas import tpu as pltpu
from jax.experimental.pallas import tpu_sc as plsc

_K_BY_HEAD = (10, 20, 40, 500)
_NW = 32
_G = 16


def _mono16(x):
    b = lax.bitcast_convert_type(x, jnp.int32)
    return jnp.where(b >= 0, b, b ^ jnp.int32(0x7FFFFFFF))


def _sc_body(att_hbm, out_hbm, in_buf, out_buf, hist,
             sem_in0, sem_in1, sem_out):
    total_rows, n = att_hbm.shape
    rows_per_w = total_rows // _NW
    ngroups = rows_per_w // _G
    wid = lax.axis_index("c") * 16 + lax.axis_index("s")
    row0 = wid * rows_per_w
    head = lax.rem(row0 // n, 4)
    k0 = jnp.where(
        head == 0, _K_BY_HEAD[0],
        jnp.where(head == 1, _K_BY_HEAD[1],
                  jnp.where(head == 2, _K_BY_HEAD[2], _K_BY_HEAD[3])))
    k0 = jnp.minimum(k0, n).astype(jnp.int32)
    sems_in = (sem_in0, sem_in1)

    iota = lax.iota(jnp.int32, 16)
    ones16 = jnp.ones((16,), jnp.int32)
    zeros16 = jnp.zeros((16,), jnp.int32)
    k16 = jnp.broadcast_to(k0, (16,))

    def in_dma(g, sl):
        return pltpu.make_async_copy(
            att_hbm.at[pl.ds(row0 + g * _G, _G)], in_buf.at[sl], sems_in[sl])

    def out_dma(g):
        return pltpu.make_async_copy(
            out_buf, out_hbm.at[pl.ds(row0 + g * _G, _G)], sem_out)

    def scan_hist(rem_k16):
        @plsc.parallel_loop(0, 256, unroll=8,
                            carry=(zeros16, zeros16, zeros16))
        def scanned(j, carry):
            suffix, cnt, gt = carry
            b = 255 - j
            h = hist[pl.ds(b * 16, 16)]
            hist[pl.ds(b * 16, 16)] = zeros16
            suffix = suffix + h
            mask = suffix >= rem_k16
            cnt = cnt + jnp.where(mask, 1, 0)
            gt = gt + jnp.where(mask, 0, h)
            return suffix, cnt, gt

        _, cnt, gt = scanned
        return cnt - 1, gt

    def per_group(g, sl):
        slv = jnp.full((16,), sl, jnp.int32)
        in_dma(g, sl).wait()

        @plsc.parallel_loop(0, n, unroll=8,
                            carry=jnp.full((16,), -3.4e38, jnp.float32))
        def m16(c, m):
            col = (iota + c) & (n - 1)
            x = plsc.load_gather(in_buf, [slv, iota, col])
            key = _mono16(x)
            bkt = (key >> 24) + 128
            plsc.addupdate_scatter(hist, [bkt * 16 + iota], ones16)
            return jnp.maximum(m, x)

        bstar, gt = scan_hist(k16)
        prefix = bstar - 128
        rem_k = k16 - gt

        def radix_pass(shift, prefix, rem_k):
            @plsc.parallel_loop(0, n, unroll=8)
            def _(c):
                col = (iota + c) & (n - 1)
                x = plsc.load_gather(in_buf, [slv, iota, col])
                key = _mono16(x)
                match = (key >> (shift + 8)) == prefix
                bkt = (key >> shift) & 0xFF
                plsc.addupdate_scatter(hist, [bkt * 16 + iota], ones16,
                                       mask=match)

            b, gt = scan_hist(rem_k)
            return (prefix << 8) | b, rem_k - gt

        prefix, rem_k = radix_pass(16, prefix, rem_k)
        prefix, rem_k = radix_pass(8, prefix, rem_k)
        t16, _ = radix_pass(0, prefix, rem_k)
        tx16 = lax.bitcast_convert_type(
            jnp.where(t16 >= 0, t16, t16 ^ jnp.int32(0x7FFFFFFF)),
            jnp.float32)

        @plsc.parallel_loop(0, n, unroll=8,
                            carry=jnp.zeros((16,), jnp.float32))
        def z16(c, z):
            col = (iota + c) & (n - 1)
            x = plsc.load_gather(in_buf, [slv, iota, col])
            e = jnp.exp(x - m16)
            em = jnp.where(x >= tx16, e, 0.0)
            plsc.store_scatter(in_buf, [slv, iota, col], em)
            return z + em

        invz = jnp.ones((16,), jnp.float32) / z16

        @pl.when(g >= 1)
        def _():
            out_dma(g - 1).wait()

        @plsc.parallel_loop(0, n, unroll=8)
        def _(c):
            col = (iota + c) & (n - 1)
            em = plsc.load_gather(in_buf, [slv, iota, col])
            plsc.store_scatter(out_buf, [iota, col], em * invz)

        out_dma(g).start()

    @plsc.parallel_loop(0, 256, unroll=8)
    def _(v):
        hist[pl.ds(v * 16, 16)] = zeros16

    in_dma(0, 0).start()
    in_dma(1, 1).start()

    def per_pair(p, _):
        for sl in (0, 1):
            g = 2 * p + sl
            per_group(g, sl)

            @pl.when(g + 2 < ngroups)
            def _():
                in_dma(g + 2, sl).start()
        return 0

    lax.fori_loop(0, ngroups // 2, per_pair, 0)
    out_dma(ngroups - 1).wait()


def kernel(attention):
    bsz, heads, n, _ = attention.shape
    att2 = attention.reshape(bsz * heads * n, n)
    mesh = plsc.VectorSubcoreMesh(core_axis_name="c", subcore_axis_name="s")
    out2 = pl.kernel(
        _sc_body,
        out_type=jax.ShapeDtypeStruct(att2.shape, att2.dtype),
        mesh=mesh,
        compiler_params=pltpu.CompilerParams(needs_layout_passes=False),
        scratch_types=[
            pltpu.VMEM((2, _G, n), jnp.float32),
            pltpu.VMEM((_G, n), jnp.float32),
            pltpu.VMEM((256 * 16,), jnp.int32),
            pltpu.SemaphoreType.DMA,
            pltpu.SemaphoreType.DMA,
            pltpu.SemaphoreType.DMA,
        ],
    )(att2)
    return out2.reshape(attention.shape)


# speedup vs baseline: 15.2965x; 1.9085x over previous
import functools

import jax
import jax.numpy as jnp
from jax import lax
from jax.experimental import pallas as pl
from jax.experimental.pallas import tpu as pltpu
from jax.experimental.pallas import tpu_sc as plsc

_K_BY_HEAD = (10, 20, 40, 500)
_NW = 32
_G = 16
_NS_SC = 12
_TC_ROW_BLOCK = 256


def _k_of_head(head, n):
    k = jnp.where(
        head == 0, _K_BY_HEAD[0],
        jnp.where(head == 1, _K_BY_HEAD[1],
                  jnp.where(head == 2, _K_BY_HEAD[2], _K_BY_HEAD[3])))
    return jnp.minimum(k, n).astype(jnp.int32)


def _mono16(x):
    b = lax.bitcast_convert_type(x, jnp.int32)
    return jnp.where(b >= 0, b, b ^ jnp.int32(0x7FFFFFFF))


def _sc_body(att_hbm, out_hbm, in_buf, out_buf, hist,
             sem_in0, sem_in1, sem_out):
    total_rows, n = att_hbm.shape
    rows_per_w = total_rows // _NW
    ngroups = rows_per_w // _G
    wid = lax.axis_index("c") * 16 + lax.axis_index("s")
    row0 = wid * rows_per_w
    sems_in = (sem_in0, sem_in1)

    iota = lax.iota(jnp.int32, 16)
    ones16 = jnp.ones((16,), jnp.int32)
    zeros16 = jnp.zeros((16,), jnp.int32)

    def in_dma(g, sl):
        return pltpu.make_async_copy(
            att_hbm.at[pl.ds(row0 + g * _G, _G)], in_buf.at[sl], sems_in[sl])

    def out_dma(g):
        return pltpu.make_async_copy(
            out_buf, out_hbm.at[pl.ds(row0 + g * _G, _G)], sem_out)

    def scan_hist(rem_k16):
        @plsc.parallel_loop(0, 256, unroll=8,
                            carry=(zeros16, zeros16, zeros16))
        def scanned(j, carry):
            suffix, cnt, gt = carry
            b = 255 - j
            h = hist[pl.ds(b * 16, 16)]
            hist[pl.ds(b * 16, 16)] = zeros16
            suffix = suffix + h
            mask = suffix >= rem_k16
            cnt = cnt + jnp.where(mask, 1, 0)
            gt = gt + jnp.where(mask, 0, h)
            return suffix, cnt, gt

        _, cnt, gt = scanned
        return cnt - 1, gt

    def per_group(g, sl):
        slv = jnp.full((16,), sl, jnp.int32)
        head = lax.rem((row0 + g * _G) // n, 4)
        k16 = jnp.broadcast_to(_k_of_head(head, n), (16,))
        in_dma(g, sl).wait()

        @plsc.parallel_loop(0, n, unroll=8,
                            carry=jnp.full((16,), -3.4e38, jnp.float32))
        def m16(c, m):
            col = (iota + c) & (n - 1)
            x = plsc.load_gather(in_buf, [slv, iota, col])
            key = _mono16(x)
            bkt = (key >> 24) + 128
            plsc.addupdate_scatter(hist, [bkt * 16 + iota], ones16)
            return jnp.maximum(m, x)

        bstar, gt = scan_hist(k16)
        prefix = bstar - 128
        rem_k = k16 - gt

        def radix_pass(shift, prefix, rem_k):
            @plsc.parallel_loop(0, n, unroll=8)
            def _(c):
                col = (iota + c) & (n - 1)
                x = plsc.load_gather(in_buf, [slv, iota, col])
                key = _mono16(x)
                match = (key >> (shift + 8)) == prefix
                bkt = (key >> shift) & 0xFF
                plsc.addupdate_scatter(hist, [bkt * 16 + iota], ones16,
                                       mask=match)

            b, gt = scan_hist(rem_k)
            return (prefix << 8) | b, rem_k - gt

        prefix, rem_k = radix_pass(16, prefix, rem_k)
        prefix, rem_k = radix_pass(8, prefix, rem_k)
        t16, _ = radix_pass(0, prefix, rem_k)
        tx16 = lax.bitcast_convert_type(
            jnp.where(t16 >= 0, t16, t16 ^ jnp.int32(0x7FFFFFFF)),
            jnp.float32)

        @plsc.parallel_loop(0, n, unroll=8,
                            carry=jnp.zeros((16,), jnp.float32))
        def z16(c, z):
            col = (iota + c) & (n - 1)
            x = plsc.load_gather(in_buf, [slv, iota, col])
            e = jnp.exp(x - m16)
            em = jnp.where(x >= tx16, e, 0.0)
            plsc.store_scatter(in_buf, [slv, iota, col], em)
            return z + em

        invz = jnp.ones((16,), jnp.float32) / z16

        @pl.when(g >= 1)
        def _():
            out_dma(g - 1).wait()

        @plsc.parallel_loop(0, n, unroll=8)
        def _(c):
            col = (iota + c) & (n - 1)
            em = plsc.load_gather(in_buf, [slv, iota, col])
            plsc.store_scatter(out_buf, [iota, col], em * invz)

        out_dma(g).start()

    @plsc.parallel_loop(0, 256, unroll=8)
    def _(v):
        hist[pl.ds(v * 16, 16)] = zeros16

    in_dma(0, 0).start()
    in_dma(1, 1).start()

    def per_pair(p, _):
        for sl in (0, 1):
            g = 2 * p + sl
            per_group(g, sl)

            @pl.when(g + 2 < ngroups)
            def _():
                in_dma(g + 2, sl).start()
        return 0

    lax.fori_loop(0, ngroups // 2, per_pair, 0)
    out_dma(ngroups - 1).wait()


def _sc_call(att2):
    rows, n = att2.shape
    mesh = plsc.VectorSubcoreMesh(core_axis_name="c", subcore_axis_name="s")
    return pl.kernel(
        _sc_body,
        out_type=jax.ShapeDtypeStruct((rows, n), att2.dtype),
        mesh=mesh,
        compiler_params=pltpu.CompilerParams(needs_layout_passes=False),
        scratch_types=[
            pltpu.VMEM((2, _G, n), jnp.float32),
            pltpu.VMEM((_G, n), jnp.float32),
            pltpu.VMEM((256 * 16,), jnp.int32),
            pltpu.SemaphoreType.DMA,
            pltpu.SemaphoreType.DMA,
            pltpu.SemaphoreType.DMA,
        ],
    )(att2)


def _tc_body(slab0, x_ref, o_ref):
    head = lax.rem(slab0 + pl.program_id(0), 4)
    x = x_ref[0]
    r, n = x.shape
    k = _k_of_head(head, n)

    key = _mono16(x)

    def count_ge(cand):
        return jnp.sum((key >= cand).astype(jnp.int32), axis=-1, keepdims=True)

    t = jnp.full((r, 1), jnp.int32(-2147483648))
    cand0 = jnp.zeros((r, 1), jnp.int32)
    t = jnp.where(count_ge(cand0) >= k, cand0, t)

    def step(i, t):
        cand = t + (jnp.int32(1) << (jnp.int32(30) - i))
        return jnp.where(count_ge(cand) >= k, cand, t)

    t = lax.fori_loop(0, 31, step, t, unroll=True)

    m = jnp.max(x, axis=-1, keepdims=True)
    e = jnp.exp(x - m)
    sel = key >= t
    z = jnp.sum(jnp.where(sel, e, 0.0), axis=-1, keepdims=True)
    o_ref[0] = jnp.where(sel, e / z, 0.0)


def _tc_call(att3, slab0):
    nslab, n, _ = att3.shape
    r = min(_TC_ROW_BLOCK, n)
    return pl.pallas_call(
        functools.partial(_tc_body, slab0),
        grid=(nslab, n // r),
        in_specs=[pl.BlockSpec((1, r, n), lambda s, i: (s, i, 0))],
        out_specs=pl.BlockSpec((1, r, n), lambda s, i: (s, i, 0)),
        out_shape=jax.ShapeDtypeStruct(att3.shape, att3.dtype),
        compiler_params=pltpu.CompilerParams(
            dimension_semantics=("parallel", "arbitrary")),
    )(att3)


def kernel(attention):
    bsz, heads, n, _ = attention.shape
    att2 = attention.reshape(bsz * heads * n, n)
    nslab = bsz * heads
    ns_sc = min(_NS_SC, nslab)
    sc_rows = ns_sc * n
    sc_out = _sc_call(att2[:sc_rows])
    tc_out = _tc_call(att2[sc_rows:].reshape(nslab - ns_sc, n, n), ns_sc)
    out2 = jnp.concatenate([sc_out, tc_out.reshape(-1, n)], axis=0)
    return out2.reshape(attention.shape)


# speedup vs baseline: 19.5235x; 1.2763x over previous
import functools

import jax
import jax.numpy as jnp
from jax import lax
from jax.experimental import pallas as pl
from jax.experimental.pallas import tpu as pltpu
from jax.experimental.pallas import tpu_sc as plsc

_K_BY_HEAD = (10, 20, 40, 500)
_NW = 32
_G = 16
_NS_SC = 12
_TC_ROW_BLOCK = 256


def _k_of_head(head, n):
    k = jnp.where(
        head == 0, _K_BY_HEAD[0],
        jnp.where(head == 1, _K_BY_HEAD[1],
                  jnp.where(head == 2, _K_BY_HEAD[2], _K_BY_HEAD[3])))
    return jnp.minimum(k, n).astype(jnp.int32)


def _mono16(x):
    b = lax.bitcast_convert_type(x, jnp.int32)
    return jnp.where(b >= 0, b, b ^ jnp.int32(0x7FFFFFFF))


def _sc_body(sc_rows, att_hbm, out_hbm, in_buf, out_buf, hist,
             sem_in0, sem_in1, sem_out):
    n = att_hbm.shape[1]
    rows_per_w = sc_rows // _NW
    ngroups = rows_per_w // _G
    wid = lax.axis_index("c") * 16 + lax.axis_index("s")
    row0 = wid * rows_per_w
    sems_in = (sem_in0, sem_in1)

    iota = lax.iota(jnp.int32, 16)
    ones16 = jnp.ones((16,), jnp.int32)
    zeros16 = jnp.zeros((16,), jnp.int32)

    def in_dma(g, sl):
        return pltpu.make_async_copy(
            att_hbm.at[pl.ds(row0 + g * _G, _G)], in_buf.at[sl], sems_in[sl])

    def out_dma(g):
        return pltpu.make_async_copy(
            out_buf, out_hbm.at[pl.ds(row0 + g * _G, _G)], sem_out)

    def scan_hist(rem_k16):
        @plsc.parallel_loop(0, 256, unroll=8,
                            carry=(zeros16, zeros16, zeros16))
        def scanned(j, carry):
            suffix, cnt, gt = carry
            b = 255 - j
            h = hist[pl.ds(b * 16, 16)]
            hist[pl.ds(b * 16, 16)] = zeros16
            suffix = suffix + h
            mask = suffix >= rem_k16
            cnt = cnt + jnp.where(mask, 1, 0)
            gt = gt + jnp.where(mask, 0, h)
            return suffix, cnt, gt

        _, cnt, gt = scanned
        return cnt - 1, gt

    def per_group(g, sl):
        slv = jnp.full((16,), sl, jnp.int32)
        head = lax.rem((row0 + g * _G) // n, 4)
        k16 = jnp.broadcast_to(_k_of_head(head, n), (16,))
        in_dma(g, sl).wait()

        @plsc.parallel_loop(0, n, unroll=8,
                            carry=jnp.full((16,), -3.4e38, jnp.float32))
        def m16(c, m):
            col = (iota + c) & (n - 1)
            x = plsc.load_gather(in_buf, [slv, iota, col])
            key = _mono16(x)
            bkt = (key >> 24) + 128
            plsc.addupdate_scatter(hist, [bkt * 16 + iota], ones16)
            return jnp.maximum(m, x)

        bstar, gt = scan_hist(k16)
        prefix = bstar - 128
        rem_k = k16 - gt

        def radix_pass(shift, prefix, rem_k):
            @plsc.parallel_loop(0, n, unroll=8)
            def _(c):
                col = (iota + c) & (n - 1)
                x = plsc.load_gather(in_buf, [slv, iota, col])
                key = _mono16(x)
                match = (key >> (shift + 8)) == prefix
                bkt = (key >> shift) & 0xFF
                plsc.addupdate_scatter(hist, [bkt * 16 + iota], ones16,
                                       mask=match)

            b, gt = scan_hist(rem_k)
            return (prefix << 8) | b, rem_k - gt

        prefix, rem_k = radix_pass(16, prefix, rem_k)
        prefix, rem_k = radix_pass(8, prefix, rem_k)
        t16, _ = radix_pass(0, prefix, rem_k)
        tx16 = lax.bitcast_convert_type(
            jnp.where(t16 >= 0, t16, t16 ^ jnp.int32(0x7FFFFFFF)),
            jnp.float32)

        @plsc.parallel_loop(0, n, unroll=8,
                            carry=jnp.zeros((16,), jnp.float32))
        def z16(c, z):
            col = (iota + c) & (n - 1)
            x = plsc.load_gather(in_buf, [slv, iota, col])
            e = jnp.exp(x - m16)
            em = jnp.where(x >= tx16, e, 0.0)
            plsc.store_scatter(in_buf, [slv, iota, col], em)
            return z + em

        invz = jnp.ones((16,), jnp.float32) / z16

        @pl.when(g >= 1)
        def _():
            out_dma(g - 1).wait()

        @plsc.parallel_loop(0, n, unroll=8)
        def _(c):
            col = (iota + c) & (n - 1)
            em = plsc.load_gather(in_buf, [slv, iota, col])
            plsc.store_scatter(out_buf, [iota, col], em * invz)

        out_dma(g).start()

    @plsc.parallel_loop(0, 256, unroll=8)
    def _(v):
        hist[pl.ds(v * 16, 16)] = zeros16

    in_dma(0, 0).start()
    in_dma(1, 1).start()

    def per_pair(p, _):
        for sl in (0, 1):
            g = 2 * p + sl
            per_group(g, sl)

            @pl.when(g + 2 < ngroups)
            def _():
                in_dma(g + 2, sl).start()
        return 0

    lax.fori_loop(0, ngroups // 2, per_pair, 0)
    out_dma(ngroups - 1).wait()


def _sc_call(att2, sc_rows):
    rows, n = att2.shape
    mesh = plsc.VectorSubcoreMesh(core_axis_name="c", subcore_axis_name="s")
    return pl.kernel(
        functools.partial(_sc_body, sc_rows),
        out_type=jax.ShapeDtypeStruct((sc_rows, n), att2.dtype),
        mesh=mesh,
        compiler_params=pltpu.CompilerParams(needs_layout_passes=False),
        scratch_types=[
            pltpu.VMEM((2, _G, n), jnp.float32),
            pltpu.VMEM((_G, n), jnp.float32),
            pltpu.VMEM((256 * 16,), jnp.int32),
            pltpu.SemaphoreType.DMA,
            pltpu.SemaphoreType.DMA,
            pltpu.SemaphoreType.DMA,
        ],
    )(att2)


def _tc_body(slab0, x_ref, o_ref):
    head = lax.rem(slab0 + pl.program_id(0), 4)
    x = x_ref[0]
    r, n = x.shape
    k = _k_of_head(head, n)

    key = _mono16(x)

    def count_ge(cand):
        return jnp.sum((key >= cand).astype(jnp.int32), axis=-1, keepdims=True)

    t = jnp.full((r, 1), jnp.int32(-2147483648))
    cand0 = jnp.zeros((r, 1), jnp.int32)
    t = jnp.where(count_ge(cand0) >= k, cand0, t)

    def step(i, t):
        cand = t + (jnp.int32(1) << (jnp.int32(30) - i))
        return jnp.where(count_ge(cand) >= k, cand, t)

    t = lax.fori_loop(0, 31, step, t, unroll=True)

    m = jnp.max(x, axis=-1, keepdims=True)
    e = jnp.exp(x - m)
    sel = key >= t
    z = jnp.sum(jnp.where(sel, e, 0.0), axis=-1, keepdims=True)
    o_ref[0] = jnp.where(sel, e / z, 0.0)


def _tc_call(att3, slab0):
    nslab, n, _ = att3.shape
    r = min(_TC_ROW_BLOCK, n)
    return pl.pallas_call(
        functools.partial(_tc_body, slab0),
        grid=(nslab - slab0, n // r),
        in_specs=[pl.BlockSpec((1, r, n), lambda s, i: (s + slab0, i, 0))],
        out_specs=pl.BlockSpec((1, r, n), lambda s, i: (s + slab0, i, 0)),
        out_shape=jax.ShapeDtypeStruct(att3.shape, att3.dtype),
        compiler_params=pltpu.CompilerParams(
            dimension_semantics=("parallel", "arbitrary")),
    )(att3)


def kernel(attention):
    bsz, heads, n, _ = attention.shape
    att2 = attention.reshape(bsz * heads * n, n)
    nslab = bsz * heads
    ns_sc = min(_NS_SC, nslab)
    sc_rows = ns_sc * n
    sc_out = _sc_call(att2, sc_rows)
    tc_full = _tc_call(attention.reshape(nslab, n, n), ns_sc)
    out2 = lax.dynamic_update_slice(tc_full.reshape(-1, n), sc_out, (0, 0))
    return out2.reshape(attention.shape)


# speedup vs baseline: 20.0853x; 1.0288x over previous
import functools

import jax
import jax.numpy as jnp
from jax import lax
from jax.experimental import pallas as pl
from jax.experimental.pallas import tpu as pltpu
from jax.experimental.pallas import tpu_sc as plsc

_K_BY_HEAD = (10, 20, 40, 500)
_NW = 32
_G = 16
_NS_SC = 11
_TC_ROW_BLOCK = 256


def _k_of_head(head, n):
    k = jnp.where(
        head == 0, _K_BY_HEAD[0],
        jnp.where(head == 1, _K_BY_HEAD[1],
                  jnp.where(head == 2, _K_BY_HEAD[2], _K_BY_HEAD[3])))
    return jnp.minimum(k, n).astype(jnp.int32)


def _mono16(x):
    b = lax.bitcast_convert_type(x, jnp.int32)
    return jnp.where(b >= 0, b, b ^ jnp.int32(0x7FFFFFFF))


def _sc_body(sc_rows, att_hbm, out_hbm, in_buf, out_buf, hist,
             sem_in0, sem_in1, sem_out):
    n = att_hbm.shape[1]
    rows_per_w = sc_rows // _NW
    ngroups = rows_per_w // _G
    wid = lax.axis_index("c") * 16 + lax.axis_index("s")
    row0 = wid * rows_per_w
    sems_in = (sem_in0, sem_in1)

    iota = lax.iota(jnp.int32, 16)
    ones16 = jnp.ones((16,), jnp.int32)
    zeros16 = jnp.zeros((16,), jnp.int32)

    def in_dma(g, sl):
        return pltpu.make_async_copy(
            att_hbm.at[pl.ds(row0 + g * _G, _G)], in_buf.at[sl], sems_in[sl])

    def out_dma(g):
        return pltpu.make_async_copy(
            out_buf, out_hbm.at[pl.ds(row0 + g * _G, _G)], sem_out)

    def scan_hist(rem_k16):
        @plsc.parallel_loop(0, 256, unroll=8,
                            carry=(zeros16, zeros16, zeros16))
        def scanned(j, carry):
            suffix, cnt, gt = carry
            b = 255 - j
            h = hist[pl.ds(b * 16, 16)]
            hist[pl.ds(b * 16, 16)] = zeros16
            suffix = suffix + h
            mask = suffix >= rem_k16
            cnt = cnt + jnp.where(mask, 1, 0)
            gt = gt + jnp.where(mask, 0, h)
            return suffix, cnt, gt

        _, cnt, gt = scanned
        return cnt - 1, gt

    def per_group(g, sl):
        slv = jnp.full((16,), sl, jnp.int32)
        head = lax.rem((row0 + g * _G) // n, 4)
        k16 = jnp.broadcast_to(_k_of_head(head, n), (16,))
        in_dma(g, sl).wait()

        @plsc.parallel_loop(0, n, unroll=8,
                            carry=jnp.full((16,), -3.4e38, jnp.float32))
        def m16(c, m):
            col = (iota + c) & (n - 1)
            x = plsc.load_gather(in_buf, [slv, iota, col])
            key = _mono16(x)
            bkt = (key >> 24) + 128
            plsc.addupdate_scatter(hist, [bkt * 16 + iota], ones16)
            return jnp.maximum(m, x)

        bstar, gt = scan_hist(k16)
        prefix = bstar - 128
        rem_k = k16 - gt

        def radix_pass(shift, prefix, rem_k):
            @plsc.parallel_loop(0, n, unroll=8)
            def _(c):
                col = (iota + c) & (n - 1)
                x = plsc.load_gather(in_buf, [slv, iota, col])
                key = _mono16(x)
                match = (key >> (shift + 8)) == prefix
                bkt = (key >> shift) & 0xFF
                plsc.addupdate_scatter(hist, [bkt * 16 + iota], ones16,
                                       mask=match)

            b, gt = scan_hist(rem_k)
            return (prefix << 8) | b, rem_k - gt

        prefix, rem_k = radix_pass(16, prefix, rem_k)
        prefix, rem_k = radix_pass(8, prefix, rem_k)
        t16, _ = radix_pass(0, prefix, rem_k)
        tx16 = lax.bitcast_convert_type(
            jnp.where(t16 >= 0, t16, t16 ^ jnp.int32(0x7FFFFFFF)),
            jnp.float32)

        @plsc.parallel_loop(0, n, unroll=8,
                            carry=jnp.zeros((16,), jnp.float32))
        def z16(c, z):
            col = (iota + c) & (n - 1)
            x = plsc.load_gather(in_buf, [slv, iota, col])
            e = jnp.exp(x - m16)
            em = jnp.where(x >= tx16, e, 0.0)
            plsc.store_scatter(in_buf, [slv, iota, col], em)
            return z + em

        invz = jnp.ones((16,), jnp.float32) / z16

        @pl.when(g >= 1)
        def _():
            out_dma(g - 1).wait()

        @plsc.parallel_loop(0, n, unroll=8)
        def _(c):
            col = (iota + c) & (n - 1)
            em = plsc.load_gather(in_buf, [slv, iota, col])
            plsc.store_scatter(out_buf, [iota, col], em * invz)

        out_dma(g).start()

    @plsc.parallel_loop(0, 256, unroll=8)
    def _(v):
        hist[pl.ds(v * 16, 16)] = zeros16

    in_dma(0, 0).start()
    in_dma(1, 1).start()

    def per_pair(p, _):
        for sl in (0, 1):
            g = 2 * p + sl
            per_group(g, sl)

            @pl.when(g + 2 < ngroups)
            def _():
                in_dma(g + 2, sl).start()
        return 0

    lax.fori_loop(0, ngroups // 2, per_pair, 0)
    out_dma(ngroups - 1).wait()


def _sc_call(att2, sc_rows):
    rows, n = att2.shape
    mesh = plsc.VectorSubcoreMesh(core_axis_name="c", subcore_axis_name="s")
    return pl.kernel(
        functools.partial(_sc_body, sc_rows),
        out_type=jax.ShapeDtypeStruct((sc_rows, n), att2.dtype),
        mesh=mesh,
        compiler_params=pltpu.CompilerParams(needs_layout_passes=False),
        scratch_types=[
            pltpu.VMEM((2, _G, n), jnp.float32),
            pltpu.VMEM((_G, n), jnp.float32),
            pltpu.VMEM((256 * 16,), jnp.int32),
            pltpu.SemaphoreType.DMA,
            pltpu.SemaphoreType.DMA,
            pltpu.SemaphoreType.DMA,
        ],
    )(att2)


def _tc_body(slab0, x_ref, o_ref):
    head = lax.rem(slab0 + pl.program_id(0), 4)
    x = x_ref[0]
    r, n = x.shape
    k = _k_of_head(head, n)

    key = _mono16(x)

    def count_ge(cand):
        return jnp.sum((key >= cand).astype(jnp.int32), axis=-1, keepdims=True)

    t = jnp.full((r, 1), jnp.int32(-2147483648))
    cand0 = jnp.zeros((r, 1), jnp.int32)
    t = jnp.where(count_ge(cand0) >= k, cand0, t)

    def step(i, t):
        cand = t + (jnp.int32(1) << (jnp.int32(30) - i))
        return jnp.where(count_ge(cand) >= k, cand, t)

    t = lax.fori_loop(0, 31, step, t, unroll=True)

    m = jnp.max(x, axis=-1, keepdims=True)
    e = jnp.exp(x - m)
    sel = key >= t
    z = jnp.sum(jnp.where(sel, e, 0.0), axis=-1, keepdims=True)
    o_ref[0] = jnp.where(sel, e / z, 0.0)


def _tc_call(att3, slab0):
    nslab, n, _ = att3.shape
    r = min(_TC_ROW_BLOCK, n)
    return pl.pallas_call(
        functools.partial(_tc_body, slab0),
        grid=(nslab - slab0, n // r),
        in_specs=[pl.BlockSpec((1, r, n), lambda s, i: (s + slab0, i, 0))],
        out_specs=pl.BlockSpec((1, r, n), lambda s, i: (s + slab0, i, 0)),
        out_shape=jax.ShapeDtypeStruct(att3.shape, att3.dtype),
        compiler_params=pltpu.CompilerParams(
            dimension_semantics=("parallel", "arbitrary")),
    )(att3)


def kernel(attention):
    bsz, heads, n, _ = attention.shape
    att2 = attention.reshape(bsz * heads * n, n)
    nslab = bsz * heads
    ns_sc = min(_NS_SC, nslab)
    sc_rows = ns_sc * n
    sc_out = _sc_call(att2, sc_rows)
    tc_full = _tc_call(attention.reshape(nslab, n, n), ns_sc)
    out2 = lax.dynamic_update_slice(tc_full.reshape(-1, n), sc_out, (0, 0))
    return out2.reshape(attention.shape)
